# baseline (speedup 1.0000x reference)
import jax
import jax.numpy as jnp
from jax import lax
from jax.experimental import pallas as pl
from jax.experimental.pallas import tpu as pltpu
from jax.experimental.pallas import tpu_sc as plsc

_NC = 2
_NS = 16
_L = 16
_B = 64
_G = 4
_R = 32


def _cheb_step(is_first, NP, CAP, H):
    NH = NP // _NS
    NFC = NH // _R
    NCH = CAP // _B
    NG = NCH // _G
    EDC = 2 * _B
    GED = _G * EDC
    GVL = _G * _B
    CL = _NS * NCH * EDC + GED
    assert NH % _R == 0 and CAP % (_G * _B) == 0 and NG % 2 == 0

    mesh = plsc.VectorSubcoreMesh(
        core_axis_name="c", subcore_axis_name="s",
        num_cores=_NC, num_subcores=_NS)

    def body(tp1, tp2, oacc_in, edata, valsH, thA, thB,
             t_out, oacc_out,
             ebuf0, ebuf1, vbuf0, vbuf1, gbuf0, gbuf1, gbuf2, gbuf3,
             pbuf, obuf, thA_v, thB_v, acc_v, sem0, sem1, sem2, sem3):
        c = lax.axis_index("c")
        s = lax.axis_index("s")
        ebufs = (ebuf0, ebuf1)
        vbufs = (vbuf0, vbuf1)
        gbufs = (gbuf0, gbuf1, gbuf2, gbuf3)
        sems = (sem0, sem1, sem2, sem3)

        def zrow(r, carry):
            for j in range(H // _L):
                acc_v[r, pl.ds(j * _L, _L)] = jnp.zeros((_L,), jnp.float32)
            return carry
        lax.fori_loop(0, NH, zrow, 0)

        def load_grp(grp, eb):
            base = c * CL + (s * NCH + grp * _G) * EDC
            pltpu.sync_copy(edata.at[pl.ds(base, GED)], ebufs[eb])
            vbase = (s * NCH + grp * _G) * _B
            pltpu.sync_copy(valsH.at[pl.ds(vbase, GVL)], vbufs[eb])

        def start_g(eb, cb):
            idx = ebufs[eb].at[pl.ds(cb * EDC, _B)]
            pltpu.async_copy(tp1.at[idx], gbufs[cb], sems[cb])

        def wait_g(cb):
            pltpu.make_async_copy(tp1.at[pl.ds(0, _B)],
                                  gbufs[cb], sems[cb]).wait()

        def compute(eb, cb):
            ebuf = ebufs[eb]
            vbuf = vbufs[eb]
            gbuf = gbufs[cb]

            def eadd(g, ecarry):
                rvec = ebuf[pl.ds(cb * EDC + _B + g * _L, _L)]
                vvec = vbuf[pl.ds(cb * _B + g * _L, _L)]
                for i2 in range(_L):
                    v = vvec[i2]
                    rr = rvec[i2]
                    e = g * _L + i2
                    for j in range(H // _L):
                        sl = pl.ds(j * _L, _L)
                        plsc.addupdate(acc_v.at[rr, sl], gbuf[e, sl] * v)
                return ecarry
            lax.fori_loop(0, _B // _L, eadd, 0)

        load_grp(0, 0)
        for cb in range(_G):
            start_g(0, cb)

        def outer(io2, carry):
            for gb in range(2):
                grp = io2 * 2 + gb
                load_grp(grp + 1, 1 - gb)
                for cb in range(_G):
                    wait_g(cb)
                    compute(gb, cb)
                    start_g(1 - gb, cb)
            return carry
        lax.fori_loop(0, NG // 2, outer, 0)
        for cb in range(_G):
            wait_g(cb)

        if is_first:
            pltpu.sync_copy(thA.at[c], thA_v)
        pltpu.sync_copy(thB.at[c], thB_v)

        def fin(ci, carry):
            rloc = pl.multiple_of(ci * _R, _R)
            g = pl.multiple_of(c * NP + s * NH + rloc, _R)
            pltpu.sync_copy(tp2.at[pl.ds(g, _R)], pbuf)
            if not is_first:
                pltpu.sync_copy(oacc_in.at[pl.ds(g, _R)], obuf)

            def frow(r, fcarry):
                for j in range(H // _L):
                    sl = pl.ds(j * _L, _L)
                    a = acc_v[rloc + r, sl]
                    if is_first:
                        t = a
                        o = thA_v[sl] * pbuf[r, sl] + thB_v[sl] * t
                    else:
                        t = 2.0 * a - pbuf[r, sl]
                        o = obuf[r, sl] + thB_v[sl] * t
                    gbuf0[r, sl] = t
                    obuf[r, sl] = o
                return fcarry
            lax.fori_loop(0, _R, frow, 0)
            pltpu.sync_copy(gbuf0.at[pl.ds(0, _R)], t_out.at[pl.ds(g, _R)])
            pltpu.sync_copy(obuf, oacc_out.at[pl.ds(g, _R)])
            return carry
        lax.fori_loop(0, NFC, fin, 0)

    out_t = [jax.ShapeDtypeStruct((_NC * NP, H), jnp.float32),
             jax.ShapeDtypeStruct((_NC * NP, H), jnp.float32)]
    scratch = [
        pltpu.VMEM((GED,), jnp.int32),
        pltpu.VMEM((GED,), jnp.int32),
        pltpu.VMEM((GVL,), jnp.float32),
        pltpu.VMEM((GVL,), jnp.float32),
        pltpu.VMEM((_B, H), jnp.float32),
        pltpu.VMEM((_B, H), jnp.float32),
        pltpu.VMEM((_B, H), jnp.float32),
        pltpu.VMEM((_B, H), jnp.float32),
        pltpu.VMEM((_R, H), jnp.float32),
        pltpu.VMEM((_R, H), jnp.float32),
        pltpu.VMEM((H,), jnp.float32),
        pltpu.VMEM((H,), jnp.float32),
        pltpu.VMEM((NP // _NS, H), jnp.float32),
        pltpu.SemaphoreType.DMA,
        pltpu.SemaphoreType.DMA,
        pltpu.SemaphoreType.DMA,
        pltpu.SemaphoreType.DMA,
    ]
    return pl.kernel(body, out_type=out_t, mesh=mesh, scratch_types=scratch,
                     name="cheb_first" if is_first else "cheb_step")


@jax.jit
def kernel(x, slap_vals, theta, slap_rows, slap_cols):
    N, D = x.shape
    K = theta.shape[0]
    E = slap_rows.shape[0]
    H = D // _NC

    NP = -(-N // (_NS * _R)) * (_NS * _R)
    NP = max(NP, -(-N // (_NS * 8)) * (_NS * 8))
    NH = NP // _NS
    mean = E * NH / N
    ALN = 2 * _G * _B
    CAP = int(-(-(mean + 9.5 * (mean ** 0.5)) // ALN) * ALN)

    bucket = slap_rows // NH
    order = jnp.argsort(bucket)
    srows = slap_rows[order]
    scols = slap_cols[order]
    svals = slap_vals[order]
    sbucket = bucket[order]
    starts = jnp.searchsorted(sbucket, jnp.arange(_NS, dtype=jnp.int32))
    counts = jnp.append(starts[1:], E) - starts

    slot = jnp.arange(_NS * CAP, dtype=jnp.int32)
    b = slot // CAP
    r = slot - b * CAP
    src = starts[b] + r
    valid = r < counts[b]
    srcc = jnp.where(valid, jnp.minimum(src, E - 1), 0)
    rowsP = jnp.where(valid, jnp.take(srows, srcc) - b * NH, 0)
    colsP = jnp.where(valid, jnp.take(scols, srcc), 0)
    valsP = jnp.where(valid, jnp.take(svals, srcc), 0.0)
    valsH = jnp.concatenate([valsP, jnp.zeros((_G * _B,), jnp.float32)])

    M = _NS * CAP // _B
    slack = jnp.zeros((_G, 2, _B), jnp.int32)
    eparts = []
    for ct in range(_NC):
        ed = jnp.stack([(colsP + ct * NP).reshape(M, _B),
                        rowsP.reshape(M, _B)], axis=1)
        eparts.append(jnp.concatenate([ed, slack]).reshape(-1))
    edata = jnp.concatenate(eparts)
    th = theta.reshape(K, _NC, H)

    x_st = x.reshape(N, _NC, H).transpose(1, 0, 2)
    x_st = jnp.pad(x_st, ((0, 0), (0, NP - N), (0, 0))).reshape(_NC * NP, H)

    first = _cheb_step(True, NP, CAP, H)
    step = _cheb_step(False, NP, CAP, H)

    t1, oacc = first(x_st, x_st, x_st, edata, valsH, th[0], th[1])
    tm2, tm1 = x_st, t1
    for k in range(2, K):
        tk, oacc = step(tm1, tm2, oacc, edata, valsH, th[k], th[k])
        tm2, tm1 = tm1, tk

    out = oacc.reshape(_NC, NP, H)[:, :N]
    return out.transpose(1, 0, 2).reshape(N, D)


# speedup vs baseline: 1.0304x; 1.0304x over previous
import jax
import jax.numpy as jnp
from jax import lax
from jax.experimental import pallas as pl
from jax.experimental.pallas import tpu as pltpu
from jax.experimental.pallas import tpu_sc as plsc

_NC = 2
_NS = 16
_NT = _NC * _NS
_L = 16
_B = 64
_G = 2
_R = 16


def _cheb_step(is_first, NP, CAP, H):
    NH = NP // _NT
    NFC = NH // _R
    NCH = CAP // _B
    NG = NCH // _G
    EDC = 2 * _B
    GED = _G * EDC
    GVL = _G * _B
    CL = _NT * NCH * EDC + GED
    assert NH % _R == 0 and CAP % (_G * _B) == 0 and NG % 2 == 0

    mesh = plsc.VectorSubcoreMesh(
        core_axis_name="c", subcore_axis_name="s",
        num_cores=_NC, num_subcores=_NS)

    def body(tp1, tp2, oacc_in, edata, valsH, thA, thB,
             t_out, oacc_out,
             ebuf0, ebuf1, vbuf0, vbuf1, gbuf0, gbuf1,
             pbuf, obuf, thA_v, thB_v, acc_v, sem0, sem1):
        c = lax.axis_index("c")
        s = lax.axis_index("s")
        t = s * _NC + c
        ebufs = (ebuf0, ebuf1)
        vbufs = (vbuf0, vbuf1)
        gbufs = (gbuf0, gbuf1)
        sems = (sem0, sem1)

        def zrow(r, carry):
            for j in range(H // _L):
                acc_v[r, pl.ds(j * _L, _L)] = jnp.zeros((_L,), jnp.float32)
            return carry
        lax.fori_loop(0, NH, zrow, 0)

        def load_grp(grp, eb):
            base = (t * NCH + grp * _G) * EDC
            pltpu.sync_copy(edata.at[pl.ds(base, GED)], ebufs[eb])
            vbase = (t * NCH + grp * _G) * _B
            pltpu.sync_copy(valsH.at[pl.ds(vbase, GVL)], vbufs[eb])

        def start_g(eb, cb):
            idx = ebufs[eb].at[pl.ds(cb * EDC, _B)]
            pltpu.async_copy(tp1.at[idx], gbufs[cb % 2], sems[cb % 2])

        def wait_g(cb):
            pltpu.make_async_copy(tp1.at[pl.ds(0, _B)],
                                  gbufs[cb % 2], sems[cb % 2]).wait()

        def compute(eb, cb):
            ebuf = ebufs[eb]
            vbuf = vbufs[eb]
            gbuf = gbufs[cb % 2]

            def eadd(g, ecarry):
                rvec = ebuf[pl.ds(cb * EDC + _B + g * _L, _L)]
                vvec = vbuf[pl.ds(cb * _B + g * _L, _L)]
                for i2 in range(_L):
                    v = vvec[i2]
                    rr = rvec[i2]
                    e = g * _L + i2
                    for j in range(H // _L):
                        sl = pl.ds(j * _L, _L)
                        plsc.addupdate(acc_v.at[rr, sl], gbuf[e, sl] * v)
                return ecarry
            lax.fori_loop(0, _B // _L, eadd, 0)

        load_grp(0, 0)
        start_g(0, 0)
        start_g(0, 1)

        def outer(io2, carry):
            for gb in range(2):
                load_grp(io2 * 2 + gb + 1, 1 - gb)
                for cb in range(_G):
                    wait_g(cb)
                    compute(gb, cb)
                    start_g(1 - gb, cb)
            return carry
        lax.fori_loop(0, NG // 2, outer, 0)
        wait_g(0)
        wait_g(1)

        if is_first:
            pltpu.sync_copy(thA, thA_v)
        pltpu.sync_copy(thB, thB_v)

        def fin(ci, carry):
            rloc = pl.multiple_of(ci * _R, _R)
            g = pl.multiple_of(t * NH + rloc, _R)
            pltpu.sync_copy(tp2.at[pl.ds(g, _R)], pbuf)
            if not is_first:
                pltpu.sync_copy(oacc_in.at[pl.ds(g, _R)], obuf)

            def frow(r, fcarry):
                for j in range(H // _L):
                    sl = pl.ds(j * _L, _L)
                    a = acc_v[rloc + r, sl]
                    if is_first:
                        tt = a
                        o = thA_v[sl] * pbuf[r, sl] + thB_v[sl] * tt
                    else:
                        tt = 2.0 * a - pbuf[r, sl]
                        o = obuf[r, sl] + thB_v[sl] * tt
                    gbuf0[r, sl] = tt
                    obuf[r, sl] = o
                return fcarry
            lax.fori_loop(0, _R, frow, 0)
            pltpu.sync_copy(gbuf0.at[pl.ds(0, _R)], t_out.at[pl.ds(g, _R)])
            pltpu.sync_copy(obuf, oacc_out.at[pl.ds(g, _R)])
            return carry
        lax.fori_loop(0, NFC, fin, 0)

    out_t = [jax.ShapeDtypeStruct((NP, H), jnp.float32),
             jax.ShapeDtypeStruct((NP, H), jnp.float32)]
    scratch = [
        pltpu.VMEM((GED,), jnp.int32),
        pltpu.VMEM((GED,), jnp.int32),
        pltpu.VMEM((GVL,), jnp.float32),
        pltpu.VMEM((GVL,), jnp.float32),
        pltpu.VMEM((_B, H), jnp.float32),
        pltpu.VMEM((_B, H), jnp.float32),
        pltpu.VMEM((_R, H), jnp.float32),
        pltpu.VMEM((_R, H), jnp.float32),
        pltpu.VMEM((H,), jnp.float32),
        pltpu.VMEM((H,), jnp.float32),
        pltpu.VMEM((NP // _NT, H), jnp.float32),
        pltpu.SemaphoreType.DMA,
        pltpu.SemaphoreType.DMA,
    ]
    return pl.kernel(body, out_type=out_t, mesh=mesh, scratch_types=scratch,
                     name="cheb_first" if is_first else "cheb_step")


@jax.jit
def kernel(x, slap_vals, theta, slap_rows, slap_cols):
    N, D = x.shape
    K = theta.shape[0]
    E = slap_rows.shape[0]
    H = D

    NP = -(-N // (_NT * _R)) * (_NT * _R)
    NH = NP // _NT
    mean = E * NH / N
    ALN = 2 * _G * _B
    CAP = int(-(-(mean + 9.5 * (mean ** 0.5)) // ALN) * ALN)

    bucket = slap_rows // NH
    order = jnp.argsort(bucket)
    srows = slap_rows[order]
    scols = slap_cols[order]
    svals = slap_vals[order]
    sbucket = bucket[order]
    starts = jnp.searchsorted(sbucket, jnp.arange(_NT, dtype=jnp.int32))
    counts = jnp.append(starts[1:], E) - starts

    slot = jnp.arange(_NT * CAP, dtype=jnp.int32)
    b = slot // CAP
    r = slot - b * CAP
    src = starts[b] + r
    valid = r < counts[b]
    srcc = jnp.where(valid, jnp.minimum(src, E - 1), 0)
    rowsP = jnp.where(valid, jnp.take(srows, srcc) - b * NH, 0)
    colsP = jnp.where(valid, jnp.take(scols, srcc), 0)
    valsP = jnp.where(valid, jnp.take(svals, srcc), 0.0)
    valsH = jnp.concatenate([valsP, jnp.zeros((_G * _B,), jnp.float32)])

    M = _NT * CAP // _B
    slack = jnp.zeros((_G, 2, _B), jnp.int32)
    ed = jnp.stack([colsP.reshape(M, _B), rowsP.reshape(M, _B)], axis=1)
    edata = jnp.concatenate([ed, slack]).reshape(-1)
    th = theta.reshape(K, H)

    x_p = jnp.pad(x, ((0, NP - N), (0, 0)))

    first = _cheb_step(True, NP, CAP, H)
    step = _cheb_step(False, NP, CAP, H)

    t1, oacc = first(x_p, x_p, x_p, edata, valsH, th[0], th[1])
    tm2, tm1 = x_p, t1
    for k in range(2, K):
        tk, oacc = step(tm1, tm2, oacc, edata, valsH, th[k], th[k])
        tm2, tm1 = tm1, tk

    return oacc[:N]


# speedup vs baseline: 1.1197x; 1.0867x over previous
import jax
import jax.numpy as jnp
from jax import lax
from jax.experimental import pallas as pl
from jax.experimental.pallas import tpu as pltpu
from jax.experimental.pallas import tpu_sc as plsc

_NC = 2
_NS = 16
_NT = _NC * _NS
_L = 16
_B = 64
_G = 2
_R = 16


def _cheb_step(is_first, NP, CAP, H):
    NH = NP // _NT
    NFC = NH // _R
    NCH = CAP // _B
    NG = NCH // _G
    EDC = 2 * _B
    GED = _G * EDC
    GVL = _G * _B
    CL = _NT * NCH * EDC + GED
    assert NH % _R == 0 and CAP % (_G * _B) == 0 and NG % 2 == 0

    mesh = plsc.VectorSubcoreMesh(
        core_axis_name="c", subcore_axis_name="s",
        num_cores=_NC, num_subcores=_NS)

    def body(tp1, tp2, oacc_in, edata, valsH, thA, thB,
             t_out, oacc_out,
             tb_out,
             ebuf0, ebuf1, vbuf0, vbuf1, gbuf0, gbuf1,
             pbuf, obuf, tbbuf, thA_v, thB_v, acc_v, sem0, sem1):
        c = lax.axis_index("c")
        s = lax.axis_index("s")
        t = s * _NC + c
        ebufs = (ebuf0, ebuf1)
        vbufs = (vbuf0, vbuf1)
        gbufs = (gbuf0, gbuf1)
        sems = (sem0, sem1)

        def zrow(r, carry):
            for j in range(H // _L):
                acc_v[r, pl.ds(j * _L, _L)] = jnp.zeros((_L,), jnp.float32)
            return carry
        lax.fori_loop(0, NH, zrow, 0)

        def load_grp(grp, eb):
            base = (t * NCH + grp * _G) * EDC
            pltpu.sync_copy(edata.at[pl.ds(base, GED)], ebufs[eb])
            vbase = (t * NCH + grp * _G) * _B
            pltpu.sync_copy(valsH.at[pl.ds(vbase, GVL)], vbufs[eb])

        def start_g(eb, cb):
            idx = ebufs[eb].at[pl.ds(cb * EDC, _B)]
            pltpu.async_copy(tp1.at[idx], gbufs[cb % 2], sems[cb % 2])

        def wait_g(cb):
            pltpu.make_async_copy(tp1.at[pl.ds(0, _B)],
                                  gbufs[cb % 2], sems[cb % 2]).wait()

        def compute(eb, cb):
            ebuf = ebufs[eb]
            vbuf = vbufs[eb]
            gbuf = gbufs[cb % 2]

            def eadd(g, ecarry):
                rvec = ebuf[pl.ds(cb * EDC + _B + g * _L, _L)]
                vvec = vbuf[pl.ds(cb * _B + g * _L, _L)]
                for i2 in range(_L):
                    v = vvec[i2]
                    rr = rvec[i2]
                    e = g * _L + i2
                    for jj in range(H // (2 * _L)):
                        w = gbuf[e, pl.ds(jj * _L, _L)]
                        ab = plsc.bitcast(w, jnp.bfloat16)
                        a0, a1 = plsc.unpack(
                            ab, format=plsc.PackFormat.INTERLEAVED)
                        plsc.addupdate(
                            acc_v.at[rr, pl.ds(2 * jj * _L, _L)], a0 * v)
                        plsc.addupdate(
                            acc_v.at[rr, pl.ds((2 * jj + 1) * _L, _L)],
                            a1 * v)
                return ecarry
            lax.fori_loop(0, _B // _L, eadd, 0)

        load_grp(0, 0)
        start_g(0, 0)
        start_g(0, 1)

        def outer(io2, carry):
            for gb in range(2):
                load_grp(io2 * 2 + gb + 1, 1 - gb)
                for cb in range(_G):
                    wait_g(cb)
                    compute(gb, cb)
                    start_g(1 - gb, cb)
            return carry
        lax.fori_loop(0, NG // 2, outer, 0)
        wait_g(0)
        wait_g(1)

        if is_first:
            pltpu.sync_copy(thA, thA_v)
        pltpu.sync_copy(thB, thB_v)

        def fin(ci, carry):
            rloc = pl.multiple_of(ci * _R, _R)
            g = pl.multiple_of(t * NH + rloc, _R)
            pltpu.sync_copy(tp2.at[pl.ds(g, _R)], pbuf)
            if not is_first:
                pltpu.sync_copy(oacc_in.at[pl.ds(g, _R)], obuf)

            def frow(r, fcarry):
                for jj in range(H // (2 * _L)):
                    tts = []
                    for j in (2 * jj, 2 * jj + 1):
                        sl = pl.ds(j * _L, _L)
                        a = acc_v[rloc + r, sl]
                        if is_first:
                            tt = a
                            o = thA_v[sl] * pbuf[r, sl] + thB_v[sl] * tt
                        else:
                            tt = 2.0 * a - pbuf[r, sl]
                            o = obuf[r, sl] + thB_v[sl] * tt
                        pbuf[r, sl] = tt
                        obuf[r, sl] = o
                        tts.append(tt)
                    packed = plsc.pack(
                        tts[0], tts[1], format=plsc.PackFormat.INTERLEAVED)
                    tbbuf[r, pl.ds(jj * _L, _L)] = plsc.bitcast(
                        packed, jnp.int32)
                return fcarry
            lax.fori_loop(0, _R, frow, 0)
            pltpu.sync_copy(pbuf, t_out.at[pl.ds(g, _R)])
            pltpu.sync_copy(tbbuf, tb_out.at[pl.ds(g, _R)])
            pltpu.sync_copy(obuf, oacc_out.at[pl.ds(g, _R)])
            return carry
        lax.fori_loop(0, NFC, fin, 0)

    out_t = [jax.ShapeDtypeStruct((NP, H), jnp.float32),
             jax.ShapeDtypeStruct((NP, H), jnp.float32),
             jax.ShapeDtypeStruct((NP, H // 2), jnp.int32)]
    scratch = [
        pltpu.VMEM((GED,), jnp.int32),
        pltpu.VMEM((GED,), jnp.int32),
        pltpu.VMEM((GVL,), jnp.float32),
        pltpu.VMEM((GVL,), jnp.float32),
        pltpu.VMEM((_B, H // 2), jnp.int32),
        pltpu.VMEM((_B, H // 2), jnp.int32),
        pltpu.VMEM((_R, H), jnp.float32),
        pltpu.VMEM((_R, H), jnp.float32),
        pltpu.VMEM((_R, H // 2), jnp.int32),
        pltpu.VMEM((H,), jnp.float32),
        pltpu.VMEM((H,), jnp.float32),
        pltpu.VMEM((NP // _NT, H), jnp.float32),
        pltpu.SemaphoreType.DMA,
        pltpu.SemaphoreType.DMA,
    ]
    return pl.kernel(body, out_type=out_t, mesh=mesh, scratch_types=scratch,
                     compiler_params=pltpu.CompilerParams(
                         needs_layout_passes=False),
                     name="cheb_first" if is_first else "cheb_step")


@jax.jit
def kernel(x, slap_vals, theta, slap_rows, slap_cols):
    N, D = x.shape
    K = theta.shape[0]
    E = slap_rows.shape[0]
    H = D

    NP = -(-N // (_NT * _R)) * (_NT * _R)
    NH = NP // _NT
    mean = E * NH / N
    ALN = 2 * _G * _B
    CAP = int(-(-(mean + 9.5 * (mean ** 0.5)) // ALN) * ALN)

    bucket = slap_rows // NH
    order = jnp.argsort(bucket)
    srows = slap_rows[order]
    scols = slap_cols[order]
    svals = slap_vals[order]
    sbucket = bucket[order]
    starts = jnp.searchsorted(sbucket, jnp.arange(_NT, dtype=jnp.int32))
    counts = jnp.append(starts[1:], E) - starts

    slot = jnp.arange(_NT * CAP, dtype=jnp.int32)
    b = slot // CAP
    r = slot - b * CAP
    src = starts[b] + r
    valid = r < counts[b]
    srcc = jnp.where(valid, jnp.minimum(src, E - 1), 0)
    rowsP = jnp.where(valid, jnp.take(srows, srcc) - b * NH, 0)
    colsP = jnp.where(valid, jnp.take(scols, srcc), 0)
    valsP = jnp.where(valid, jnp.take(svals, srcc), 0.0)
    valsH = jnp.concatenate([valsP, jnp.zeros((_G * _B,), jnp.float32)])

    M = _NT * CAP // _B
    slack = jnp.zeros((_G, 2, _B), jnp.int32)
    ed = jnp.stack([colsP.reshape(M, _B), rowsP.reshape(M, _B)], axis=1)
    edata = jnp.concatenate([ed, slack]).reshape(-1)
    th = theta.reshape(K, H)

    x_p = jnp.pad(x, ((0, NP - N), (0, 0)))
    xb = x_p.reshape(NP, H // 32, 2, _L).swapaxes(2, 3)
    xb = xb.astype(jnp.bfloat16).reshape(NP, H // 2, 2)
    xb = lax.bitcast_convert_type(xb, jnp.int32)

    first = _cheb_step(True, NP, CAP, H)
    step = _cheb_step(False, NP, CAP, H)

    t1, oacc, t1b = first(xb, x_p, x_p, edata, valsH, th[0], th[1])
    tm2, tm1b = x_p, t1b
    tm1 = t1
    for k in range(2, K):
        tk, oacc, tkb = step(tm1b, tm2, oacc, edata, valsH, th[k], th[k])
        tm2, tm1, tm1b = tm1, tk, tkb

    return oacc[:N]
